# baseline (device time: 29086 ns/iter reference)
import jax
import jax.numpy as jnp
from jax import lax
from jax.experimental import pallas as pl
from jax.experimental.pallas import tpu as pltpu

N_DEV = 4
DH = 64
B, SQ, D = 2, 256, 768
HL = 8
KVL = 2
M = B * SQ
HD = HL * DH


def _fused_kernel(x2d, Wq, Wk, Wv, Wo):
    HALF, QUART, CW = M // 2, M // 4, D // 2
    bf = jnp.bfloat16

    def body(x_ref, wq_ref, wk_ref, wv_ref, wo_ref, out_ref,
             buf_ref, o_buf, wq_b, wk_l, wv_l, wo_b,
             c1a, c1b, c2a, c2b, send_sems, recv_sems):
        i = lax.axis_index("i")
        pa = i ^ 1
        pb = 3 - i
        k1a = (i + 1) // 2 % 2
        k2a = i // 2
        k1b = i // 2
        k2b = i % 2

        barrier = pltpu.get_barrier_semaphore()
        for nbr in (pa, pb):
            pl.semaphore_signal(
                barrier, inc=1,
                device_id=(nbr,), device_id_type=pl.DeviceIdType.MESH,
            )
        pl.semaphore_wait(barrier, 2)

        wq_b[...] = wq_ref[...].astype(bf)
        wk_l[...] = wk_ref[:, pl.ds(i * KVL * DH, KVL * DH)].astype(bf)
        wv_l[...] = wv_ref[:, pl.ds(i * KVL * DH, KVL * DH)].astype(bf)
        wo_b[...] = wo_ref[...].astype(bf)

        def attn_batch(rbase):
            rows = pl.ds(rbase, SQ)
            xq = x_ref[rows, :].astype(bf)
            qb = jnp.dot(
                xq, wq_b[...], preferred_element_type=jnp.float32
            ).astype(bf)
            kb = jnp.dot(
                xq, wk_l[...], preferred_element_type=jnp.float32
            ).astype(bf)
            vb = jnp.dot(
                xq, wv_l[...], preferred_element_type=jnp.float32
            ).astype(bf)
            for h in range(HL):
                k = h // 4
                qh = qb[:, h * DH:(h + 1) * DH]
                kh = kb[:, k * DH:(k + 1) * DH]
                vh = vb[:, k * DH:(k + 1) * DH]
                s = lax.dot_general(
                    qh, kh, (((1,), (1,)), ((), ())),
                    preferred_element_type=jnp.float32,
                ) * 0.125
                p = jnp.exp(s - jnp.max(s, axis=1, keepdims=True))
                l = jnp.sum(p, axis=1, keepdims=True)
                pv = jnp.dot(
                    p.astype(bf), vh, preferred_element_type=jnp.float32
                )
                o_buf[rows, h * DH:(h + 1) * DH] = (pv / l).astype(bf)

        attn_batch(0)
        attn_batch(SQ)

        buf_ref[...] = jnp.dot(
            o_buf[...], wo_b[...], preferred_element_type=jnp.float32
        ).astype(bf)

        def xchg(slot, partner, r0, nr, c0, dst_comm=None):
            src = buf_ref.at[pl.ds(r0, nr), pl.ds(c0, CW)]
            dst = src if dst_comm is None else dst_comm
            r = pltpu.make_async_remote_copy(
                src_ref=src, dst_ref=dst,
                send_sem=send_sems.at[slot], recv_sem=recv_sems.at[slot],
                device_id=(partner,), device_id_type=pl.DeviceIdType.MESH,
            )
            r.start()
            return r

        def acc(r0, nr, c0, comm):
            rows, cols = pl.ds(r0, nr), pl.ds(c0, CW)
            buf_ref[rows, cols] = buf_ref[rows, cols] + comm[...]

        ra = xchg(0, pa, (1 - k1a) * HALF, HALF, 0, c1a)
        rb = xchg(1, pb, (1 - k1b) * HALF, HALF, CW, c1b)
        ra.wait()
        acc(k1a * HALF, HALF, 0, c1a)
        rb.wait()
        acc(k1b * HALF, HALF, CW, c1b)

        ra = xchg(2, pb, (2 * k1a + 1 - k2a) * QUART, QUART, 0, c2a)
        rb = xchg(3, pa, (2 * k1b + 1 - k2b) * QUART, QUART, CW, c2b)
        ra.wait()
        acc((2 * k1a + k2a) * QUART, QUART, 0, c2a)
        rb.wait()
        acc((2 * k1b + k2b) * QUART, QUART, CW, c2b)

        ra = xchg(4, pb, (2 * k1a + k2a) * QUART, QUART, 0)
        rb = xchg(5, pa, (2 * k1b + k2b) * QUART, QUART, CW)
        ra.wait()
        rb.wait()

        ra = xchg(6, pa, k1a * HALF, HALF, 0)
        rb = xchg(7, pb, k1b * HALF, HALF, CW)
        ra.wait()
        rb.wait()

        out_ref[...] = buf_ref[...].astype(jnp.float32)

    return pl.pallas_call(
        body,
        out_shape=jax.ShapeDtypeStruct((M, D), jnp.float32),
        in_specs=[pl.BlockSpec(memory_space=pltpu.VMEM)] * 5,
        out_specs=pl.BlockSpec(memory_space=pltpu.VMEM),
        scratch_shapes=[
            pltpu.VMEM((M, D), bf),
            pltpu.VMEM((M, HD), bf),
            pltpu.VMEM((D, HD), bf),
            pltpu.VMEM((D, KVL * DH), bf),
            pltpu.VMEM((D, KVL * DH), bf),
            pltpu.VMEM((HD, D), bf),
            pltpu.VMEM((HALF, CW), bf),
            pltpu.VMEM((HALF, CW), bf),
            pltpu.VMEM((QUART, CW), bf),
            pltpu.VMEM((QUART, CW), bf),
            pltpu.SemaphoreType.DMA((8,)),
            pltpu.SemaphoreType.DMA((8,)),
        ],
        compiler_params=pltpu.CompilerParams(collective_id=0),
    )(x2d, Wq, Wk, Wv, Wo)


def kernel(x, Wq, Wo, Wk, Wv):
    out = _fused_kernel(x.reshape(M, D), Wq, Wk, Wv, Wo)
    return out.reshape(B, SQ, D)


# device time: 28735 ns/iter; 1.0122x vs baseline; 1.0122x over previous
import jax
import jax.numpy as jnp
from jax import lax
from jax.experimental import pallas as pl
from jax.experimental.pallas import tpu as pltpu

N_DEV = 4
DH = 64
B, SQ, D = 2, 256, 768
HL = 8
KVL = 2
M = B * SQ
HD = HL * DH


def _fused_kernel(x2d, Wq, Wk, Wv, Wo):
    HALF, QUART, CW = M // 2, M // 4, D // 2
    bf = jnp.bfloat16

    def body(x_ref, wq_ref, wk_ref, wv_ref, wo_ref, out_ref,
             buf_ref, o_buf, wq_b, wk_l, wv_l, wo_b,
             c1a, c1b, c2a, c2b, send_sems, recv_sems):
        i = lax.axis_index("i")
        pa = i ^ 1
        pb = 3 - i
        k1a = (i + 1) // 2 % 2
        k2a = i // 2
        k1b = i // 2
        k2b = i % 2

        barrier = pltpu.get_barrier_semaphore()
        for nbr in (pa, pb):
            pl.semaphore_signal(
                barrier, inc=1,
                device_id=(nbr,), device_id_type=pl.DeviceIdType.MESH,
            )

        wq_b[...] = wq_ref[...].astype(bf)
        wk_l[...] = wk_ref[:, pl.ds(i * KVL * DH, KVL * DH)].astype(bf)
        wv_l[...] = wv_ref[:, pl.ds(i * KVL * DH, KVL * DH)].astype(bf)
        wo_b[...] = wo_ref[...].astype(bf)

        def attn_batch(rbase):
            rows = pl.ds(rbase, SQ)
            xq = x_ref[rows, :].astype(bf)
            qb = jnp.dot(
                xq, wq_b[...], preferred_element_type=jnp.float32
            ).astype(bf)
            kb = jnp.dot(
                xq, wk_l[...], preferred_element_type=jnp.float32
            ).astype(bf)
            vb = jnp.dot(
                xq, wv_l[...], preferred_element_type=jnp.float32
            ).astype(bf)
            for h in range(HL):
                k = h // 4
                qh = qb[:, h * DH:(h + 1) * DH]
                kh = kb[:, k * DH:(k + 1) * DH]
                vh = vb[:, k * DH:(k + 1) * DH]
                s = lax.dot_general(
                    qh, kh, (((1,), (1,)), ((), ())),
                    preferred_element_type=jnp.float32,
                ) * 0.125
                p = jnp.exp(s - jnp.max(s, axis=1, keepdims=True))
                l = jnp.sum(p, axis=1, keepdims=True)
                pv = jnp.dot(
                    p.astype(bf), vh, preferred_element_type=jnp.float32
                )
                o_buf[rows, h * DH:(h + 1) * DH] = (pv / l).astype(bf)

        def wo_mm(r0, c0):
            buf_ref[pl.ds(r0, HALF), pl.ds(c0, CW)] = jnp.dot(
                o_buf[pl.ds(r0, HALF), :], wo_b[:, pl.ds(c0, CW)],
                preferred_element_type=jnp.float32,
            ).astype(bf)

        def xchg(slot, partner, r0, nr, c0, dst_comm=None):
            src = buf_ref.at[pl.ds(r0, nr), pl.ds(c0, CW)]
            dst = src if dst_comm is None else dst_comm
            r = pltpu.make_async_remote_copy(
                src_ref=src, dst_ref=dst,
                send_sem=send_sems.at[slot], recv_sem=recv_sems.at[slot],
                device_id=(partner,), device_id_type=pl.DeviceIdType.MESH,
            )
            r.start()
            return r

        def acc(r0, nr, c0, comm):
            rows, cols = pl.ds(r0, nr), pl.ds(c0, CW)
            buf_ref[rows, cols] = buf_ref[rows, cols] + comm[...]

        def cast_out(r0, nr, c0):
            rows, cols = pl.ds(r0, nr), pl.ds(c0, CW)
            out_ref[rows, cols] = buf_ref[rows, cols].astype(jnp.float32)

        attn_batch((1 - k1a) * SQ)
        wo_mm((1 - k1a) * HALF, 0)
        pl.semaphore_wait(barrier, 2)
        ra = xchg(0, pa, (1 - k1a) * HALF, HALF, 0, c1a)
        attn_batch(k1a * SQ)
        wo_mm((1 - k1b) * HALF, CW)
        rb = xchg(1, pb, (1 - k1b) * HALF, HALF, CW, c1b)
        wo_mm(k1a * HALF, 0)
        wo_mm(k1b * HALF, CW)
        ra.wait()
        acc(k1a * HALF, HALF, 0, c1a)
        rb.wait()
        acc(k1b * HALF, HALF, CW, c1b)

        ra = xchg(2, pb, (2 * k1a + 1 - k2a) * QUART, QUART, 0, c2a)
        rb = xchg(3, pa, (2 * k1b + 1 - k2b) * QUART, QUART, CW, c2b)
        ra.wait()
        acc((2 * k1a + k2a) * QUART, QUART, 0, c2a)
        rb.wait()
        acc((2 * k1b + k2b) * QUART, QUART, CW, c2b)

        ra = xchg(4, pb, (2 * k1a + k2a) * QUART, QUART, 0)
        rb = xchg(5, pa, (2 * k1b + k2b) * QUART, QUART, CW)
        cast_out((2 * k1a + k2a) * QUART, QUART, 0)
        cast_out((2 * k1b + k2b) * QUART, QUART, CW)
        ra.wait()
        rb.wait()

        ra = xchg(6, pa, k1a * HALF, HALF, 0)
        rb = xchg(7, pb, k1b * HALF, HALF, CW)
        cast_out((2 * k1a + 1 - k2a) * QUART, QUART, 0)
        cast_out((2 * k1b + 1 - k2b) * QUART, QUART, CW)
        ra.wait()
        rb.wait()
        cast_out((1 - k1a) * HALF, HALF, 0)
        cast_out((1 - k1b) * HALF, HALF, CW)

    return pl.pallas_call(
        body,
        out_shape=jax.ShapeDtypeStruct((M, D), jnp.float32),
        in_specs=[pl.BlockSpec(memory_space=pltpu.VMEM)] * 5,
        out_specs=pl.BlockSpec(memory_space=pltpu.VMEM),
        scratch_shapes=[
            pltpu.VMEM((M, D), bf),
            pltpu.VMEM((M, HD), bf),
            pltpu.VMEM((D, HD), bf),
            pltpu.VMEM((D, KVL * DH), bf),
            pltpu.VMEM((D, KVL * DH), bf),
            pltpu.VMEM((HD, D), bf),
            pltpu.VMEM((HALF, CW), bf),
            pltpu.VMEM((HALF, CW), bf),
            pltpu.VMEM((QUART, CW), bf),
            pltpu.VMEM((QUART, CW), bf),
            pltpu.SemaphoreType.DMA((8,)),
            pltpu.SemaphoreType.DMA((8,)),
        ],
        compiler_params=pltpu.CompilerParams(collective_id=0),
    )(x2d, Wq, Wk, Wv, Wo)


def kernel(x, Wq, Wo, Wk, Wv):
    out = _fused_kernel(x.reshape(M, D), Wq, Wk, Wv, Wo)
    return out.reshape(B, SQ, D)


# device time: 27091 ns/iter; 1.0736x vs baseline; 1.0607x over previous
import jax
import jax.numpy as jnp
from jax import lax
from jax.experimental import pallas as pl
from jax.experimental.pallas import tpu as pltpu

N_DEV = 4
DH = 64
B, SQ, D = 2, 256, 768
HL = 8
KVL = 2
M = B * SQ
HD = HL * DH


def _fused_kernel(x2d, Wq, Wk, Wv, Wo):
    HALF, QUART, CW = M // 2, M // 4, D // 2
    bf = jnp.bfloat16

    def body(x_hbm, wq_hbm, wk_hbm, wv_hbm, wo_hbm, out_ref,
             x_v, wq_v, wkl_v, wvl_v, wo_v,
             buf_ref, o_buf, wq_b, wk_l, wv_l, wo_b,
             c1a, c1b, c2a, c2b, dma_sems, send_sems, recv_sems):
        i = lax.axis_index("i")
        pa = i ^ 1
        pb = 3 - i
        k1a = (i + 1) // 2 % 2
        k2a = i // 2
        k1b = i // 2
        k2b = i % 2

        cp_x = pltpu.make_async_copy(x_hbm, x_v, dma_sems.at[0])
        cp_q = pltpu.make_async_copy(wq_hbm, wq_v, dma_sems.at[1])
        cp_k = pltpu.make_async_copy(
            wk_hbm.at[:, pl.ds(i * KVL * DH, KVL * DH)], wkl_v, dma_sems.at[2]
        )
        cp_v = pltpu.make_async_copy(
            wv_hbm.at[:, pl.ds(i * KVL * DH, KVL * DH)], wvl_v, dma_sems.at[3]
        )
        cp_o = pltpu.make_async_copy(wo_hbm, wo_v, dma_sems.at[4])
        cp_x.start()
        cp_q.start()
        cp_k.start()
        cp_v.start()
        cp_o.start()

        barrier = pltpu.get_barrier_semaphore()
        for nbr in (pa, pb):
            pl.semaphore_signal(
                barrier, inc=1,
                device_id=(nbr,), device_id_type=pl.DeviceIdType.MESH,
            )

        cp_q.wait()
        wq_b[...] = wq_v[...].astype(bf)
        cp_k.wait()
        wk_l[...] = wkl_v[...].astype(bf)
        cp_v.wait()
        wv_l[...] = wvl_v[...].astype(bf)
        cp_x.wait()

        def attn_batch(rbase):
            rows = pl.ds(rbase, SQ)
            xq = x_v[rows, :].astype(bf)
            qb = jnp.dot(
                xq, wq_b[...], preferred_element_type=jnp.float32
            ).astype(bf)
            kb = jnp.dot(
                xq, wk_l[...], preferred_element_type=jnp.float32
            ).astype(bf)
            vb = jnp.dot(
                xq, wv_l[...], preferred_element_type=jnp.float32
            ).astype(bf)
            for h in range(HL):
                k = h // 4
                qh = qb[:, h * DH:(h + 1) * DH]
                kh = kb[:, k * DH:(k + 1) * DH]
                vh = vb[:, k * DH:(k + 1) * DH]
                s = lax.dot_general(
                    qh, kh, (((1,), (1,)), ((), ())),
                    preferred_element_type=jnp.float32,
                ) * 0.125
                p = jnp.exp(s - jnp.max(s, axis=1, keepdims=True))
                l = jnp.sum(p, axis=1, keepdims=True)
                pv = jnp.dot(
                    p.astype(bf), vh, preferred_element_type=jnp.float32
                )
                o_buf[rows, h * DH:(h + 1) * DH] = (pv / l).astype(bf)

        def wo_mm(r0, c0):
            buf_ref[pl.ds(r0, HALF), pl.ds(c0, CW)] = jnp.dot(
                o_buf[pl.ds(r0, HALF), :], wo_b[:, pl.ds(c0, CW)],
                preferred_element_type=jnp.float32,
            ).astype(bf)

        def xchg(slot, partner, r0, nr, c0, dst_comm=None):
            src = buf_ref.at[pl.ds(r0, nr), pl.ds(c0, CW)]
            dst = src if dst_comm is None else dst_comm
            r = pltpu.make_async_remote_copy(
                src_ref=src, dst_ref=dst,
                send_sem=send_sems.at[slot], recv_sem=recv_sems.at[slot],
                device_id=(partner,), device_id_type=pl.DeviceIdType.MESH,
            )
            r.start()
            return r

        def acc(r0, nr, c0, comm):
            rows, cols = pl.ds(r0, nr), pl.ds(c0, CW)
            buf_ref[rows, cols] = buf_ref[rows, cols] + comm[...]

        def cast_out(r0, nr, c0):
            rows, cols = pl.ds(r0, nr), pl.ds(c0, CW)
            out_ref[rows, cols] = buf_ref[rows, cols].astype(jnp.float32)

        attn_batch((1 - k1a) * SQ)
        cp_o.wait()
        wo_b[...] = wo_v[...].astype(bf)
        wo_mm((1 - k1a) * HALF, 0)
        pl.semaphore_wait(barrier, 2)
        ra = xchg(0, pa, (1 - k1a) * HALF, HALF, 0, c1a)
        attn_batch(k1a * SQ)
        wo_mm((1 - k1b) * HALF, CW)
        rb = xchg(1, pb, (1 - k1b) * HALF, HALF, CW, c1b)
        wo_mm(k1a * HALF, 0)
        wo_mm(k1b * HALF, CW)

        ra.wait()
        acc(k1a * HALF, HALF, 0, c1a)
        ra = xchg(2, pb, (2 * k1a + 1 - k2a) * QUART, QUART, 0, c2a)
        rb.wait()
        acc(k1b * HALF, HALF, CW, c1b)
        rb = xchg(3, pa, (2 * k1b + 1 - k2b) * QUART, QUART, CW, c2b)

        ra.wait()
        acc((2 * k1a + k2a) * QUART, QUART, 0, c2a)
        ra = xchg(4, pb, (2 * k1a + k2a) * QUART, QUART, 0)
        rb.wait()
        acc((2 * k1b + k2b) * QUART, QUART, CW, c2b)
        rb = xchg(5, pa, (2 * k1b + k2b) * QUART, QUART, CW)
        cast_out((2 * k1a + k2a) * QUART, QUART, 0)
        cast_out((2 * k1b + k2b) * QUART, QUART, CW)

        ra.wait()
        ra = xchg(6, pa, k1a * HALF, HALF, 0)
        cast_out((2 * k1a + 1 - k2a) * QUART, QUART, 0)
        rb.wait()
        rb = xchg(7, pb, k1b * HALF, HALF, CW)
        cast_out((2 * k1b + 1 - k2b) * QUART, QUART, CW)

        ra.wait()
        cast_out((1 - k1a) * HALF, HALF, 0)
        rb.wait()
        cast_out((1 - k1b) * HALF, HALF, CW)

    hbm = pl.BlockSpec(memory_space=pltpu.MemorySpace.HBM)
    return pl.pallas_call(
        body,
        out_shape=jax.ShapeDtypeStruct((M, D), jnp.float32),
        in_specs=[hbm] * 5,
        out_specs=pl.BlockSpec(memory_space=pltpu.VMEM),
        scratch_shapes=[
            pltpu.VMEM((M, D), jnp.float32),
            pltpu.VMEM((D, HD), jnp.float32),
            pltpu.VMEM((D, KVL * DH), jnp.float32),
            pltpu.VMEM((D, KVL * DH), jnp.float32),
            pltpu.VMEM((HD, D), jnp.float32),
            pltpu.VMEM((M, D), bf),
            pltpu.VMEM((M, HD), bf),
            pltpu.VMEM((D, HD), bf),
            pltpu.VMEM((D, KVL * DH), bf),
            pltpu.VMEM((D, KVL * DH), bf),
            pltpu.VMEM((HD, D), bf),
            pltpu.VMEM((HALF, CW), bf),
            pltpu.VMEM((HALF, CW), bf),
            pltpu.VMEM((QUART, CW), bf),
            pltpu.VMEM((QUART, CW), bf),
            pltpu.SemaphoreType.DMA((5,)),
            pltpu.SemaphoreType.DMA((8,)),
            pltpu.SemaphoreType.DMA((8,)),
        ],
        compiler_params=pltpu.CompilerParams(collective_id=0),
    )(x2d, Wq, Wk, Wv, Wo)


def kernel(x, Wq, Wo, Wk, Wv):
    out = _fused_kernel(x.reshape(M, D), Wq, Wk, Wv, Wo)
    return out.reshape(B, SQ, D)


# device time: 25620 ns/iter; 1.1353x vs baseline; 1.0574x over previous
import jax
import jax.numpy as jnp
from jax import lax
from jax.experimental import pallas as pl
from jax.experimental.pallas import tpu as pltpu

N_DEV = 4
DH = 64
B, SQ, D = 2, 256, 768
HL = 8
KVL = 2
M = B * SQ
HD = HL * DH


def _fused_kernel(x2d, Wq, Wk, Wv, Wo):
    HALF, QUART, CW = M // 2, M // 4, D // 2
    bf = jnp.bfloat16

    def body(x_hbm, wq_hbm, wk_hbm, wv_hbm, wo_hbm, out_ref,
             x_v, wq_b, wk_l, wv_l, wo_b,
             buf_ref, o_buf,
             c1a, c1b, c2a, c2b, dma_sems, send_sems, recv_sems):
        i = lax.axis_index("i")
        pa = i ^ 1
        pb = 3 - i
        k1a = (i + 1) // 2 % 2
        k2a = i // 2
        k1b = i // 2
        k2b = i % 2

        cp_x = pltpu.make_async_copy(x_hbm, x_v, dma_sems.at[0])
        cp_q = pltpu.make_async_copy(wq_hbm, wq_b, dma_sems.at[1])
        cp_k = pltpu.make_async_copy(
            wk_hbm.at[:, pl.ds(i * KVL * DH, KVL * DH)], wk_l, dma_sems.at[2]
        )
        cp_v = pltpu.make_async_copy(
            wv_hbm.at[:, pl.ds(i * KVL * DH, KVL * DH)], wv_l, dma_sems.at[3]
        )
        cp_o = pltpu.make_async_copy(wo_hbm, wo_b, dma_sems.at[4])
        cp_x.start()
        cp_q.start()
        cp_k.start()
        cp_v.start()
        cp_o.start()

        barrier = pltpu.get_barrier_semaphore()
        for nbr in (pa, pb):
            pl.semaphore_signal(
                barrier, inc=1,
                device_id=(nbr,), device_id_type=pl.DeviceIdType.MESH,
            )

        cp_x.wait()
        cp_q.wait()
        cp_k.wait()
        cp_v.wait()

        def attn_batch(rbase):
            rows = pl.ds(rbase, SQ)
            xq = x_v[rows, :]
            qb = jnp.dot(
                xq, wq_b[...], preferred_element_type=jnp.float32
            ).astype(bf)
            kb = jnp.dot(
                xq, wk_l[...], preferred_element_type=jnp.float32
            ).astype(bf)
            vb = jnp.dot(
                xq, wv_l[...], preferred_element_type=jnp.float32
            ).astype(bf)
            for h in range(HL):
                k = h // 4
                qh = qb[:, h * DH:(h + 1) * DH]
                kh = kb[:, k * DH:(k + 1) * DH]
                vh = vb[:, k * DH:(k + 1) * DH]
                s = lax.dot_general(
                    qh, kh, (((1,), (1,)), ((), ())),
                    preferred_element_type=jnp.float32,
                ) * 0.125
                p = jnp.exp(s - jnp.max(s, axis=1, keepdims=True))
                l = jnp.sum(p, axis=1, keepdims=True)
                pv = jnp.dot(
                    p.astype(bf), vh, preferred_element_type=jnp.float32
                )
                o_buf[rows, h * DH:(h + 1) * DH] = (pv / l).astype(bf)

        def wo_mm(r0, c0):
            buf_ref[pl.ds(r0, HALF), pl.ds(c0, CW)] = jnp.dot(
                o_buf[pl.ds(r0, HALF), :], wo_b[:, pl.ds(c0, CW)],
                preferred_element_type=jnp.float32,
            ).astype(bf)

        def xchg(slot, partner, r0, nr, c0, dst_comm=None):
            src = buf_ref.at[pl.ds(r0, nr), pl.ds(c0, CW)]
            dst = src if dst_comm is None else dst_comm
            r = pltpu.make_async_remote_copy(
                src_ref=src, dst_ref=dst,
                send_sem=send_sems.at[slot], recv_sem=recv_sems.at[slot],
                device_id=(partner,), device_id_type=pl.DeviceIdType.MESH,
            )
            r.start()
            return r

        def acc(r0, nr, c0, comm):
            rows, cols = pl.ds(r0, nr), pl.ds(c0, CW)
            buf_ref[rows, cols] = buf_ref[rows, cols] + comm[...]

        def cast_out(r0, nr, c0):
            rows, cols = pl.ds(r0, nr), pl.ds(c0, CW)
            out_ref[rows, cols] = buf_ref[rows, cols].astype(jnp.float32)

        attn_batch((1 - k1a) * SQ)
        cp_o.wait()
        wo_mm((1 - k1a) * HALF, 0)
        pl.semaphore_wait(barrier, 2)
        ra = xchg(0, pa, (1 - k1a) * HALF, HALF, 0, c1a)
        attn_batch(k1a * SQ)
        wo_mm((1 - k1b) * HALF, CW)
        rb = xchg(1, pb, (1 - k1b) * HALF, HALF, CW, c1b)
        wo_mm(k1a * HALF, 0)
        wo_mm(k1b * HALF, CW)

        ra.wait()
        acc(k1a * HALF, HALF, 0, c1a)
        ra = xchg(2, pb, (2 * k1a + 1 - k2a) * QUART, QUART, 0, c2a)
        rb.wait()
        acc(k1b * HALF, HALF, CW, c1b)
        rb = xchg(3, pa, (2 * k1b + 1 - k2b) * QUART, QUART, CW, c2b)

        ra.wait()
        acc((2 * k1a + k2a) * QUART, QUART, 0, c2a)
        ra = xchg(4, pb, (2 * k1a + k2a) * QUART, QUART, 0)
        rb.wait()
        acc((2 * k1b + k2b) * QUART, QUART, CW, c2b)
        rb = xchg(5, pa, (2 * k1b + k2b) * QUART, QUART, CW)
        cast_out((2 * k1a + k2a) * QUART, QUART, 0)
        cast_out((2 * k1b + k2b) * QUART, QUART, CW)

        ra.wait()
        ra = xchg(6, pa, k1a * HALF, HALF, 0)
        cast_out((2 * k1a + 1 - k2a) * QUART, QUART, 0)
        rb.wait()
        rb = xchg(7, pb, k1b * HALF, HALF, CW)
        cast_out((2 * k1b + 1 - k2b) * QUART, QUART, CW)

        ra.wait()
        cast_out((1 - k1a) * HALF, HALF, 0)
        rb.wait()
        cast_out((1 - k1b) * HALF, HALF, CW)

    hbm = pl.BlockSpec(memory_space=pltpu.MemorySpace.HBM)
    return pl.pallas_call(
        body,
        out_shape=jax.ShapeDtypeStruct((M, D), jnp.float32),
        in_specs=[hbm] * 5,
        out_specs=pl.BlockSpec(memory_space=pltpu.VMEM),
        scratch_shapes=[
            pltpu.VMEM((M, D), bf),
            pltpu.VMEM((D, HD), bf),
            pltpu.VMEM((D, KVL * DH), bf),
            pltpu.VMEM((D, KVL * DH), bf),
            pltpu.VMEM((HD, D), bf),
            pltpu.VMEM((M, D), bf),
            pltpu.VMEM((M, HD), bf),
            pltpu.VMEM((HALF, CW), bf),
            pltpu.VMEM((HALF, CW), bf),
            pltpu.VMEM((QUART, CW), bf),
            pltpu.VMEM((QUART, CW), bf),
            pltpu.SemaphoreType.DMA((5,)),
            pltpu.SemaphoreType.DMA((8,)),
            pltpu.SemaphoreType.DMA((8,)),
        ],
        compiler_params=pltpu.CompilerParams(collective_id=0),
    )(x2d, Wq, Wk, Wv, Wo)


def kernel(x, Wq, Wo, Wk, Wv):
    bf = jnp.bfloat16
    out = _fused_kernel(
        x.astype(bf).reshape(M, D),
        Wq.astype(bf), Wk.astype(bf), Wv.astype(bf), Wo.astype(bf),
    )
    return out.reshape(B, SQ, D)
